# Initial kernel scaffold; baseline (speedup 1.0000x reference)
import jax
import jax.numpy as jnp
from jax.experimental import pallas as pl


def kernel(x, edge_index, W1, b1, W2, b2, prelu_a):
    raise NotImplementedError("write your pallas kernel here")


# speedup vs baseline: 10.2586x; 10.2586x over previous
import functools

import jax
import jax.numpy as jnp
from jax import lax
from jax.experimental import pallas as pl
from jax.experimental.pallas import tpu as pltpu
from jax.experimental.pallas import tpu_sc as plsc

N = 10000
E = 160000
D = 256
NSUB = 16
NCORE = 2
CHUNK = 128
NCHUNK = 79
EPT = NCHUNK * CHUNK
EPAD = NSUB * EPT
NPAD = 10240
NACC = 10240
ROWS_PER_TILE = NACC // NSUB
DEG_PER_TILE = NPAD // NSUB
TRASH = N
BLK = 1000
HALF = D // 2

_mesh = plsc.VectorSubcoreMesh(
    core_axis_name="c", subcore_axis_name="s",
    num_cores=NCORE, num_subcores=NSUB)

_f32 = jnp.float32


def _zero_vmem_2d(ref, nrows, ncols):
    z = jnp.zeros((16,), _f32)

    def body(i, _):
        def inner(j, _):
            ref[i, pl.ds(j * 16, 16)] = z
            return 0
        return lax.fori_loop(0, ncols // 16, inner, 0)
    lax.fori_loop(0, nrows, body, 0)


def _fill_vmem_1d(ref, n, val):
    v = jnp.full((16,), val, _f32)

    def body(j, _):
        ref[pl.ds(j * 16, 16)] = v
        return 0
    lax.fori_loop(0, n // 16, body, 0)


@functools.partial(
    pl.kernel,
    out_type=(jax.ShapeDtypeStruct((NPAD,), _f32),
              jax.ShapeDtypeStruct((NPAD,), _f32)),
    mesh=_mesh,
    scratch_types=[
        pltpu.VMEM_SHARED((NPAD,), _f32),
        pltpu.VMEM((NCHUNK, CHUNK), jnp.int32),
        pltpu.VMEM((CHUNK,), _f32),
        pltpu.VMEM((DEG_PER_TILE,), _f32),
    ],
)
def _deg_kernel(rowd3, cold3, rdeg, cdeg, acc, idx, ones, zb):
    c = lax.axis_index("c")
    s = lax.axis_index("s")
    _fill_vmem_1d(ones, CHUNK, 1.0)
    _fill_vmem_1d(zb, DEG_PER_TILE, 0.0)
    pltpu.sync_copy(zb, acc.at[pl.ds(s * DEG_PER_TILE, DEG_PER_TILE)])

    def run(ei3, out):
        pltpu.sync_copy(ei3.at[s], idx)
        plsc.subcore_barrier()

        def step(i, _):
            pltpu.sync_copy(ones, acc.at[idx.at[i]], add=True)
            return 0
        lax.fori_loop(0, NCHUNK, step, 0)
        plsc.subcore_barrier()
        sl = pl.ds(s * DEG_PER_TILE, DEG_PER_TILE)
        pltpu.sync_copy(acc.at[sl], out.at[sl])

    @pl.when(c == 0)
    def _():
        run(rowd3, rdeg)

    @pl.when(c == 1)
    def _():
        run(cold3, cdeg)


@functools.partial(
    pl.kernel,
    out_type=(jax.ShapeDtypeStruct((NACC, HALF), _f32),
              jax.ShapeDtypeStruct((NACC, HALF), _f32)),
    mesh=_mesh,
    scratch_types=[
        pltpu.VMEM_SHARED((NACC, HALF), _f32),
        pltpu.VMEM((NCHUNK, CHUNK), jnp.int32),
        pltpu.VMEM((NCHUNK, CHUNK), jnp.int32),
        pltpu.VMEM((CHUNK, HALF), _f32),
    ],
)
def _spmm_kernel(g_lo, g_hi, rowg3, cold3, s_lo, s_hi,
                 acc, idxr, idxc, rows):
    c = lax.axis_index("c")
    s = lax.axis_index("s")
    zrows = ROWS_PER_TILE // 5
    _zero_vmem_2d(rows, zrows, HALF)
    base = s * ROWS_PER_TILE

    def zstep(k, _):
        pltpu.sync_copy(rows.at[pl.ds(0, zrows)],
                        acc.at[pl.ds(base + k * zrows, zrows)])
        return 0
    lax.fori_loop(0, 5, zstep, 0)

    def run(g_ref, out_ref):
        pltpu.sync_copy(rowg3.at[s], idxr)
        pltpu.sync_copy(cold3.at[s], idxc)
        plsc.subcore_barrier()

        def step(i, _):
            pltpu.sync_copy(g_ref.at[idxr.at[i]], rows)
            pltpu.sync_copy(rows, acc.at[idxc.at[i]], add=True)
            return 0
        lax.fori_loop(0, NCHUNK, step, 0)
        plsc.subcore_barrier()
        sl = pl.ds(s * ROWS_PER_TILE, ROWS_PER_TILE)
        pltpu.sync_copy(acc.at[sl], out_ref.at[sl])

    @pl.when(c == 0)
    def _():
        run(g_lo, s_lo)

    @pl.when(c == 1)
    def _():
        run(g_hi, s_hi)


def _mm1_body(x_ref, w_ref, b_ref, rd_ref, lo_ref, hi_ref):
    p = lax.dot_general(x_ref[...], w_ref[...], (((1,), (1,)), ((), ())),
                        preferred_element_type=_f32) + b_ref[...]
    g = p * lax.rsqrt(rd_ref[...] + 1.0)
    lo_ref[...] = g[:, :HALF]
    hi_ref[...] = g[:, HALF:]


def _mm2_body(slo_ref, shi_ref, glo_ref, ghi_ref, cd_ref, rd_ref,
              w_ref, b_ref, a_ref, lo_ref, hi_ref):
    cdr = cd_ref[...] + 1.0
    cs = lax.rsqrt(cdr) / cdr
    h = jnp.concatenate([slo_ref[...] + glo_ref[...],
                         shi_ref[...] + ghi_ref[...]], axis=1) * cs
    h = jnp.where(h >= 0, h, a_ref[0, 0] * h)
    p = lax.dot_general(h, w_ref[...], (((1,), (1,)), ((), ())),
                        preferred_element_type=_f32) + b_ref[...]
    g = p * lax.rsqrt(rd_ref[...] + 1.0)
    lo_ref[...] = g[:, :HALF]
    hi_ref[...] = g[:, HALF:]


def _mm3_body(slo_ref, shi_ref, glo_ref, ghi_ref, cd_ref, z_ref):
    cdr = cd_ref[...] + 1.0
    cs = lax.rsqrt(cdr) / cdr
    z_ref[...] = jnp.concatenate([slo_ref[...] + glo_ref[...],
                                  shi_ref[...] + ghi_ref[...]], axis=1) * cs


def _blk(shape, imap):
    return pl.BlockSpec(shape, imap)


_row = lambda i: (i, 0)
_rep = lambda i: (0, 0)

_mm1 = pl.pallas_call(
    _mm1_body,
    grid=(N // BLK,),
    in_specs=[_blk((BLK, D), _row), _blk((D, D), _rep), _blk((1, D), _rep),
              _blk((BLK, 1), _row)],
    out_specs=[_blk((BLK, HALF), _row), _blk((BLK, HALF), _row)],
    out_shape=(jax.ShapeDtypeStruct((N, HALF), _f32),
               jax.ShapeDtypeStruct((N, HALF), _f32)),
)

_mm2 = pl.pallas_call(
    _mm2_body,
    grid=(N // BLK,),
    in_specs=[_blk((BLK, HALF), _row)] * 4
             + [_blk((BLK, 1), _row), _blk((BLK, 1), _row),
                _blk((D, D), _rep), _blk((1, D), _rep), _blk((1, 1), _rep)],
    out_specs=[_blk((BLK, HALF), _row), _blk((BLK, HALF), _row)],
    out_shape=(jax.ShapeDtypeStruct((N, HALF), _f32),
               jax.ShapeDtypeStruct((N, HALF), _f32)),
)

_mm3 = pl.pallas_call(
    _mm3_body,
    grid=(N // BLK,),
    in_specs=[_blk((BLK, HALF), _row)] * 4 + [_blk((BLK, 1), _row)],
    out_specs=_blk((BLK, D), _row),
    out_shape=jax.ShapeDtypeStruct((N, D), _f32),
)


def kernel(x, edge_index, W1, b1, W2, b2, prelu_a):
    row = edge_index[0].astype(jnp.int32)
    col = edge_index[1].astype(jnp.int32)
    pad = EPAD - E
    rowd3 = jnp.concatenate([row, jnp.full((pad,), TRASH, jnp.int32)]
                            ).reshape(NSUB, NCHUNK, CHUNK)
    cold3 = jnp.concatenate([col, jnp.full((pad,), TRASH, jnp.int32)]
                            ).reshape(NSUB, NCHUNK, CHUNK)
    rowg3 = jnp.concatenate([row, jnp.zeros((pad,), jnp.int32)]
                            ).reshape(NSUB, NCHUNK, CHUNK)

    rdeg, cdeg = _deg_kernel(rowd3, cold3)
    rd = rdeg[:N].reshape(N, 1)
    cd = cdeg[:N].reshape(N, 1)

    b1r = b1.reshape(1, D)
    b2r = b2.reshape(1, D)
    ar = prelu_a.reshape(1, 1)

    g1lo, g1hi = _mm1(x, W1, b1r, rd)
    s1lo, s1hi = _spmm_kernel(g1lo, g1hi, rowg3, cold3)
    g2lo, g2hi = _mm2(s1lo, s1hi, g1lo, g1hi, cd, rd, W2, b2r, ar)
    s2lo, s2hi = _spmm_kernel(g2lo, g2hi, rowg3, cold3)
    return _mm3(s2lo, s2hi, g2lo, g2hi, cd)
